# baseline (speedup 1.0000x reference)
import functools

import jax
import jax.numpy as jnp
from jax import lax
from jax.experimental import pallas as pl
from jax.experimental.pallas import tpu as pltpu
from jax.experimental.pallas import tpu_sc as plsc

BATCH = 4096
SEQ_LEN = 200
EMBED_DIM = 64
VOCAB = 97655

_INFO = plsc.get_sparse_core_info()
NC = _INFO.num_cores
NS = _INFO.num_subcores
NW = NC * NS

LANES = 128
SB = BATCH // LANES
ST = SEQ_LEN // 8
NBLOCKS = SEQ_LEN * SB
BLOCKS_PER_WORKER = NBLOCKS // NW
_DO_TRANSPOSE = True
_DO_GATHER = True
_DO_WB = True
GBUF = 4
WBUF = 4
DT = EMBED_DIM // 8


def _body(idx_hbm, table_hbm, out_hbm, idx_all, *scratch):
    rows = scratch[:GBUF]
    tiles = scratch[GBUF:GBUF + WBUF]
    gsem = scratch[GBUF + WBUF:2 * GBUF + WBUF]
    wsem = scratch[2 * GBUF + WBUF:2 * GBUF + 2 * WBUF]

    wid = lax.axis_index("s") * NC + lax.axis_index("c")
    blk0 = wid * BLOCKS_PER_WORKER

    pltpu.sync_copy(idx_hbm.at[pl.ds(blk0, BLOCKS_PER_WORKER), :], idx_all)

    iota16 = lax.iota(jnp.int32, 16)
    zero16 = iota16 * 0

    def fire_gather(c, b):
        if _DO_GATHER:
            pltpu.async_copy(table_hbm.at[idx_all.at[c]], rows[b], gsem[b])

    def drain_gather(b):
        if _DO_GATHER:
            pltpu.make_async_copy(
                table_hbm.at[pl.ds(0, LANES), :], rows[b], gsem[b]).wait()

    def fire_wb(c, t):
        if not _DO_WB:
            return
        r = blk0 + c
        st = r // (SB * 8)
        bt = (r // 8) % SB
        s = st * 8 + r % 8
        pltpu.async_copy(tiles[t], out_hbm.at[s, :, bt, :, :], wsem[t])

    def drain_wb(t):
        if _DO_WB:
            pltpu.make_async_copy(
                out_hbm.at[0, :, 0, :, :], tiles[t], wsem[t]).wait()

    def transpose(b, t):
        rows_flat = rows[b].at[0]
        tiles_flat = tiles[t].at[0, 0]
        for j in range(16):
            rot = (iota16 + j) & 15
            vj = iota16 * EMBED_DIM + rot
            wj = rot * LANES + iota16

            @plsc.parallel_loop(0, 32, unroll=2)
            def _(sub):
                k0 = (sub & 7) << 4
                d0 = (sub >> 3) << 4
                v = plsc.load_gather(rows_flat, [vj + (k0 * EMBED_DIM + d0)])
                plsc.store_scatter(tiles_flat, [wj + (d0 * LANES + k0)], v)

    for b in range(GBUF):
        fire_gather(b, b)
    for t in range(WBUF):
        fire_wb(t, t)

    @pl.loop(0, BLOCKS_PER_WORKER, step=GBUF)
    def _(c0):
        for j in range(GBUF):
            c = c0 + j
            b = j
            t = j % WBUF
            drain_gather(b)
            drain_wb(t)
            if _DO_TRANSPOSE:
                transpose(b, t)
            fire_wb(c, t)
            cn = c + GBUF
            fire_gather(jnp.where(cn >= BLOCKS_PER_WORKER,
                                  cn - BLOCKS_PER_WORKER, cn), b)

    for b in range(GBUF):
        drain_gather(b)
    for t in range(WBUF):
        drain_wb(t)


@jax.jit
def _sc_gather(idx2d, table):
    mesh = plsc.VectorSubcoreMesh(core_axis_name="c", subcore_axis_name="s")
    run = functools.partial(
        pl.kernel,
        out_type=jax.ShapeDtypeStruct((SEQ_LEN, DT, SB, 8, LANES),
                                      jnp.float32),
        mesh=mesh,
        compiler_params=pltpu.CompilerParams(use_tc_tiling_on_sc=False,
                                             needs_layout_passes=False),
        scratch_types=[
            pltpu.VMEM((BLOCKS_PER_WORKER, LANES), jnp.int32),
            *[pltpu.VMEM((LANES, EMBED_DIM), jnp.float32)
              for _ in range(GBUF)],
            *[pltpu.VMEM((DT, 8, LANES), jnp.float32) for _ in range(WBUF)],
            *[pltpu.SemaphoreType.DMA for _ in range(GBUF + WBUF)],
        ],
    )(_body)
    return run(idx2d, table)


def kernel(indices, table):
    idx4 = indices.astype(jnp.int32).reshape(SB, LANES, ST, 8)
    idx2d = idx4.transpose(2, 0, 3, 1).reshape(NBLOCKS, LANES)
    out5 = _sc_gather(idx2d, table)
    return out5.transpose(2, 4, 0, 1, 3).reshape(BATCH, SEQ_LEN, EMBED_DIM)


# speedup vs baseline: 1.1671x; 1.1671x over previous
import functools

import jax
import jax.numpy as jnp
from jax import lax
from jax.experimental import pallas as pl
from jax.experimental.pallas import tpu as pltpu
from jax.experimental.pallas import tpu_sc as plsc

BATCH = 4096
SEQ_LEN = 200
EMBED_DIM = 64
VOCAB = 97655

_INFO = plsc.get_sparse_core_info()
NC = _INFO.num_cores
NS = _INFO.num_subcores
NW = NC * NS

LANES = 128
SB = BATCH // LANES
ST = SEQ_LEN // 8
NBLOCKS = SEQ_LEN * SB
BLOCKS_PER_WORKER = NBLOCKS // NW
_DO_TRANSPOSE = True
_DO_GATHER = True
_DO_WB = True
GBUF = 4
WBUF = 4
DT = EMBED_DIM // 8


def _body(idx_hbm, table_hbm, out_hbm, idx_all, *scratch):
    rows = scratch[:GBUF]
    tiles = scratch[GBUF:GBUF + WBUF]
    gsem = scratch[GBUF + WBUF:2 * GBUF + WBUF]
    wsem = scratch[2 * GBUF + WBUF:2 * GBUF + 2 * WBUF]

    wid = lax.axis_index("s") * NC + lax.axis_index("c")
    blk0 = wid * BLOCKS_PER_WORKER

    pltpu.sync_copy(idx_hbm.at[pl.ds(blk0, BLOCKS_PER_WORKER), :], idx_all)

    iota16 = lax.iota(jnp.int32, 16)
    zero16 = iota16 * 0

    def fire_gather(c, b):
        if _DO_GATHER:
            pltpu.async_copy(table_hbm.at[idx_all.at[c]], rows[b], gsem[b])

    def drain_gather(b):
        if _DO_GATHER:
            pltpu.make_async_copy(
                table_hbm.at[pl.ds(0, LANES), :], rows[b], gsem[b]).wait()

    def fire_wb(c, t):
        if not _DO_WB:
            return
        r = blk0 + c
        st = r // (SB * 8)
        bt = (r // 8) % SB
        s = st * 8 + r % 8
        pltpu.async_copy(tiles[t], out_hbm.at[s, :, bt, :, :], wsem[t])

    def drain_wb(t):
        if _DO_WB:
            pltpu.make_async_copy(
                out_hbm.at[0, :, 0, :, :], tiles[t], wsem[t]).wait()

    def transpose(b, t):
        rows_flat = rows[b].at[0]
        tiles_flat = tiles[t].at[0, 0]
        for j in range(16):
            rot = (iota16 + j) & 15
            vj = iota16 * EMBED_DIM + rot
            wj = rot * LANES + iota16

            @plsc.parallel_loop(0, 32, unroll=4)
            def _(sub):
                k0 = (sub & 7) << 4
                d0 = (sub >> 3) << 4
                v = plsc.load_gather(rows_flat, [vj + (k0 * EMBED_DIM + d0)])
                plsc.store_scatter(tiles_flat, [wj + (d0 * LANES + k0)], v)

    for b in range(GBUF):
        fire_gather(b, b)
    for t in range(WBUF):
        fire_wb(t, t)

    @pl.loop(0, BLOCKS_PER_WORKER, step=GBUF)
    def _(c0):
        for j in range(GBUF):
            c = c0 + j
            b = j
            t = j % WBUF
            drain_gather(b)
            drain_wb(t)
            if _DO_TRANSPOSE:
                transpose(b, t)
            fire_wb(c, t)
            cn = c + GBUF
            fire_gather(jnp.where(cn >= BLOCKS_PER_WORKER,
                                  cn - BLOCKS_PER_WORKER, cn), b)

    for b in range(GBUF):
        drain_gather(b)
    for t in range(WBUF):
        drain_wb(t)


@jax.jit
def _sc_gather(idx2d, table):
    mesh = plsc.VectorSubcoreMesh(core_axis_name="c", subcore_axis_name="s")
    run = functools.partial(
        pl.kernel,
        out_type=jax.ShapeDtypeStruct((SEQ_LEN, DT, SB, 8, LANES),
                                      jnp.float32),
        mesh=mesh,
        compiler_params=pltpu.CompilerParams(use_tc_tiling_on_sc=False,
                                             needs_layout_passes=False),
        scratch_types=[
            pltpu.VMEM((BLOCKS_PER_WORKER, LANES), jnp.int32),
            *[pltpu.VMEM((LANES, EMBED_DIM), jnp.float32)
              for _ in range(GBUF)],
            *[pltpu.VMEM((DT, 8, LANES), jnp.float32) for _ in range(WBUF)],
            *[pltpu.SemaphoreType.DMA for _ in range(GBUF + WBUF)],
        ],
    )(_body)
    return run(idx2d, table)


def kernel(indices, table):
    idx4 = indices.astype(jnp.int32).reshape(SB, LANES, ST, 8)
    idx2d = idx4.transpose(2, 0, 3, 1).reshape(NBLOCKS, LANES)
    out5 = _sc_gather(idx2d, table)
    return out5.transpose(2, 4, 0, 1, 3).reshape(BATCH, SEQ_LEN, EMBED_DIM)


# speedup vs baseline: 1.3749x; 1.1781x over previous
import functools

import jax
import jax.numpy as jnp
from jax import lax
from jax.experimental import pallas as pl
from jax.experimental.pallas import tpu as pltpu
from jax.experimental.pallas import tpu_sc as plsc

BATCH = 4096
SEQ_LEN = 200
EMBED_DIM = 64
VOCAB = 97655

_INFO = plsc.get_sparse_core_info()
NC = _INFO.num_cores
NS = _INFO.num_subcores
NW = NC * NS

LANES = 128
SB = BATCH // LANES
ST = SEQ_LEN // 8
NBLOCKS = SEQ_LEN * SB
BLOCKS_PER_WORKER = NBLOCKS // NW
_DO_TRANSPOSE = True
_DO_GATHER = True
_DO_WB = True
GBUF = 4
WBUF = 4
DT = EMBED_DIM // 8


def _body(idx_hbm, table_hbm, out_hbm, idx_all, *scratch):
    rows = scratch[:GBUF]
    tiles = scratch[GBUF:GBUF + WBUF]
    gsem = scratch[GBUF + WBUF:2 * GBUF + WBUF]
    wsem = scratch[2 * GBUF + WBUF:2 * GBUF + 2 * WBUF]

    wid = lax.axis_index("s") * NC + lax.axis_index("c")
    blk0 = wid * BLOCKS_PER_WORKER

    pltpu.sync_copy(idx_hbm.at[pl.ds(blk0, BLOCKS_PER_WORKER), :], idx_all)

    iota16 = lax.iota(jnp.int32, 16)
    zero16 = iota16 * 0

    def fire_gather(c, b):
        if _DO_GATHER:
            pltpu.async_copy(table_hbm.at[idx_all.at[c]], rows[b], gsem[b])

    def drain_gather(b):
        if _DO_GATHER:
            pltpu.make_async_copy(
                table_hbm.at[pl.ds(0, LANES), :], rows[b], gsem[b]).wait()

    def fire_wb(c, t):
        if not _DO_WB:
            return
        r = blk0 + c
        st = r // (SB * 8)
        bt = (r // 8) % SB
        s = st * 8 + r % 8
        pltpu.async_copy(tiles[t], out_hbm.at[s, :, bt, :, :], wsem[t])

    def drain_wb(t):
        if _DO_WB:
            pltpu.make_async_copy(
                out_hbm.at[0, :, 0, :, :], tiles[t], wsem[t]).wait()

    def transpose(b, t):
        rows_flat = rows[b].at[0]
        tiles_flat = tiles[t].at[0, 0]
        for j in range(16):
            rot = (iota16 + j) & 15
            vj = iota16 * EMBED_DIM + rot
            wj = rot * LANES + iota16

            @plsc.parallel_loop(0, 32, unroll=8)
            def _(sub):
                k0 = (sub & 7) << 4
                d0 = (sub >> 3) << 4
                v = plsc.load_gather(rows_flat, [vj + (k0 * EMBED_DIM + d0)])
                plsc.store_scatter(tiles_flat, [wj + (d0 * LANES + k0)], v)

    for b in range(GBUF):
        fire_gather(b, b)
    for t in range(WBUF):
        fire_wb(t, t)

    @pl.loop(0, BLOCKS_PER_WORKER, step=GBUF)
    def _(c0):
        for j in range(GBUF):
            c = c0 + j
            b = j
            t = j % WBUF
            drain_gather(b)
            drain_wb(t)
            if _DO_TRANSPOSE:
                transpose(b, t)
            fire_wb(c, t)
            cn = c + GBUF
            fire_gather(jnp.where(cn >= BLOCKS_PER_WORKER,
                                  cn - BLOCKS_PER_WORKER, cn), b)

    for b in range(GBUF):
        drain_gather(b)
    for t in range(WBUF):
        drain_wb(t)


@jax.jit
def _sc_gather(idx2d, table):
    mesh = plsc.VectorSubcoreMesh(core_axis_name="c", subcore_axis_name="s")
    run = functools.partial(
        pl.kernel,
        out_type=jax.ShapeDtypeStruct((SEQ_LEN, DT, SB, 8, LANES),
                                      jnp.float32),
        mesh=mesh,
        compiler_params=pltpu.CompilerParams(use_tc_tiling_on_sc=False,
                                             needs_layout_passes=False),
        scratch_types=[
            pltpu.VMEM((BLOCKS_PER_WORKER, LANES), jnp.int32),
            *[pltpu.VMEM((LANES, EMBED_DIM), jnp.float32)
              for _ in range(GBUF)],
            *[pltpu.VMEM((DT, 8, LANES), jnp.float32) for _ in range(WBUF)],
            *[pltpu.SemaphoreType.DMA for _ in range(GBUF + WBUF)],
        ],
    )(_body)
    return run(idx2d, table)


def kernel(indices, table):
    idx4 = indices.astype(jnp.int32).reshape(SB, LANES, ST, 8)
    idx2d = idx4.transpose(2, 0, 3, 1).reshape(NBLOCKS, LANES)
    out5 = _sc_gather(idx2d, table)
    return out5.transpose(2, 4, 0, 1, 3).reshape(BATCH, SEQ_LEN, EMBED_DIM)


# speedup vs baseline: 1.4120x; 1.0270x over previous
import functools

import jax
import jax.numpy as jnp
from jax import lax
from jax.experimental import pallas as pl
from jax.experimental.pallas import tpu as pltpu
from jax.experimental.pallas import tpu_sc as plsc

BATCH = 4096
SEQ_LEN = 200
EMBED_DIM = 64
VOCAB = 97655

_INFO = plsc.get_sparse_core_info()
NC = _INFO.num_cores
NS = _INFO.num_subcores
NW = NC * NS

LANES = 128
SB = BATCH // LANES
ST = SEQ_LEN // 8
NBLOCKS = SEQ_LEN * SB
BLOCKS_PER_WORKER = NBLOCKS // NW
_DO_TRANSPOSE = True
_DO_GATHER = True
_DO_WB = True
GBUF = 4
WBUF = 4
DT = EMBED_DIM // 8


def _body(idx_hbm, table_hbm, out_hbm, idx_all, *scratch):
    rows = scratch[:GBUF]
    tiles = scratch[GBUF:GBUF + WBUF]
    gsem = scratch[GBUF + WBUF:2 * GBUF + WBUF]
    wsem = scratch[2 * GBUF + WBUF:2 * GBUF + 2 * WBUF]

    wid = lax.axis_index("s") * NC + lax.axis_index("c")
    blk0 = wid * BLOCKS_PER_WORKER

    pltpu.sync_copy(idx_hbm.at[pl.ds(blk0, BLOCKS_PER_WORKER), :], idx_all)

    iota16 = lax.iota(jnp.int32, 16)
    zero16 = iota16 * 0

    def fire_gather(c, b):
        if _DO_GATHER:
            pltpu.async_copy(table_hbm.at[idx_all.at[c]], rows[b], gsem[b])

    def drain_gather(b):
        if _DO_GATHER:
            pltpu.make_async_copy(
                table_hbm.at[pl.ds(0, LANES), :], rows[b], gsem[b]).wait()

    def fire_wb(c, t):
        if not _DO_WB:
            return
        r = blk0 + c
        st = r // (SB * 8)
        bt = (r // 8) % SB
        s = st * 8 + r % 8
        pltpu.async_copy(tiles[t], out_hbm.at[s, :, bt, :, :], wsem[t])

    def drain_wb(t):
        if _DO_WB:
            pltpu.make_async_copy(
                out_hbm.at[0, :, 0, :, :], tiles[t], wsem[t]).wait()

    def transpose(b, t):
        rows_flat = rows[b].at[0]
        tiles_flat = tiles[t].at[0, 0]
        for j in range(16):
            rot = (iota16 + j) & 15
            vj = iota16 * EMBED_DIM + rot
            wj = rot * LANES + iota16

            @plsc.parallel_loop(0, 32, unroll=16)
            def _(sub):
                k0 = (sub & 7) << 4
                d0 = (sub >> 3) << 4
                v = plsc.load_gather(rows_flat, [vj + (k0 * EMBED_DIM + d0)])
                plsc.store_scatter(tiles_flat, [wj + (d0 * LANES + k0)], v)

    for b in range(GBUF):
        fire_gather(b, b)
    for t in range(WBUF):
        fire_wb(t, t)

    @pl.loop(0, BLOCKS_PER_WORKER, step=GBUF)
    def _(c0):
        for j in range(GBUF):
            c = c0 + j
            b = j
            t = j % WBUF
            drain_gather(b)
            drain_wb(t)
            if _DO_TRANSPOSE:
                transpose(b, t)
            fire_wb(c, t)
            cn = c + GBUF
            fire_gather(jnp.where(cn >= BLOCKS_PER_WORKER,
                                  cn - BLOCKS_PER_WORKER, cn), b)

    for b in range(GBUF):
        drain_gather(b)
    for t in range(WBUF):
        drain_wb(t)


@jax.jit
def _sc_gather(idx2d, table):
    mesh = plsc.VectorSubcoreMesh(core_axis_name="c", subcore_axis_name="s")
    run = functools.partial(
        pl.kernel,
        out_type=jax.ShapeDtypeStruct((SEQ_LEN, DT, SB, 8, LANES),
                                      jnp.float32),
        mesh=mesh,
        compiler_params=pltpu.CompilerParams(use_tc_tiling_on_sc=False,
                                             needs_layout_passes=False),
        scratch_types=[
            pltpu.VMEM((BLOCKS_PER_WORKER, LANES), jnp.int32),
            *[pltpu.VMEM((LANES, EMBED_DIM), jnp.float32)
              for _ in range(GBUF)],
            *[pltpu.VMEM((DT, 8, LANES), jnp.float32) for _ in range(WBUF)],
            *[pltpu.SemaphoreType.DMA for _ in range(GBUF + WBUF)],
        ],
    )(_body)
    return run(idx2d, table)


def kernel(indices, table):
    idx4 = indices.astype(jnp.int32).reshape(SB, LANES, ST, 8)
    idx2d = idx4.transpose(2, 0, 3, 1).reshape(NBLOCKS, LANES)
    out5 = _sc_gather(idx2d, table)
    return out5.transpose(2, 4, 0, 1, 3).reshape(BATCH, SEQ_LEN, EMBED_DIM)
